# Initial kernel scaffold; baseline (speedup 1.0000x reference)
import jax
import jax.numpy as jnp
from jax.experimental import pallas as pl


def kernel(h_V, batch_id, W1, b1, W2, b2):
    raise NotImplementedError("write your pallas kernel here")


# speedup vs baseline: 3.7169x; 3.7169x over previous
import functools

import jax
import jax.numpy as jnp
from jax import lax
from jax.experimental import pallas as pl
from jax.experimental.pallas import tpu as pltpu
from jax.experimental.pallas import tpu_sc as plsc

N = 320000
D = 128
S = 10000

NC = 2
NS = 16
NW = NC * NS

SL = 624
S_TAIL = S - SL * NS


P0_ROWS = 2048
P0_SUB = P0_ROWS // 128
P0_BLOCKS = N // P0_ROWS
P0_ITERS = (N // P0_ROWS + NW) // NW


def _p0_body(ids_hbm, zcnt_hbm, cnt_out, ones_v, idx_v, cnt_sh):
    cid = lax.axis_index("c")
    sid = lax.axis_index("s")
    wid = cid * NS + sid

    sb = pl.multiple_of(sid * SL, 8)
    pltpu.sync_copy(zcnt_hbm.at[pl.ds(sb, SL)], cnt_sh.at[pl.ds(sb, SL)])

    @pl.when(sid == NS - 1)
    def _():
        tb = SL * NS
        pltpu.sync_copy(zcnt_hbm.at[pl.ds(tb, S_TAIL)], cnt_sh.at[pl.ds(tb, S_TAIL)])

    def _fill(i, _):
        ones_v[i, :] = jnp.full((16,), 1.0, jnp.float32)
        return 0
    lax.fori_loop(0, 128, _fill, 0)
    plsc.subcore_barrier()

    nblocks = N // P0_ROWS

    def _block(k, _):
        b = wid + NW * k

        @pl.when(b < nblocks)
        def _():
            rb = pl.multiple_of(b * P0_SUB, 8)
            pltpu.sync_copy(ids_hbm.at[pl.ds(rb, P0_SUB)], idx_v)
            for j in range(P0_SUB):
                pltpu.sync_copy(ones_v, cnt_sh.at[idx_v.at[j]], add=True)
        return 0

    lax.fori_loop(0, P0_ITERS, _block, 0)

    @pl.when(wid == 0)
    def _():
        tail = (N - nblocks * P0_ROWS) // 128
        pltpu.sync_copy(ids_hbm.at[pl.ds(nblocks * P0_SUB, tail)],
                        idx_v.at[pl.ds(0, tail)])
        for j in range(tail):
            pltpu.sync_copy(ones_v, cnt_sh.at[idx_v.at[j]], add=True)

    plsc.subcore_barrier()
    pltpu.sync_copy(cnt_sh.at[pl.ds(sb, SL)], cnt_out.at[cid, pl.ds(sb, SL)])

    @pl.when(sid == NS - 1)
    def _():
        tb = SL * NS
        pltpu.sync_copy(cnt_sh.at[pl.ds(tb, S_TAIL)],
                        cnt_out.at[cid, pl.ds(tb, S_TAIL)])


_phase0 = functools.partial(
    pl.kernel,
    out_type=jax.ShapeDtypeStruct((NC, S, 16), jnp.float32),
    mesh=plsc.VectorSubcoreMesh(core_axis_name="c", subcore_axis_name="s"),
    scratch_types=[
        pltpu.VMEM((128, 16), jnp.float32),
        pltpu.VMEM((P0_SUB, 128), jnp.int32),
        pltpu.VMEM_SHARED((S, 16), jnp.float32),
    ],
)(_p0_body)


P1_ROWS = 256
P1_SUB = P1_ROWS // 128
P1_BLOCKS = N // P1_ROWS
P1_ITERS = (P1_BLOCKS + NW - 1) // NW


def _p1_body(h_hbm, ids_hbm, zsum_hbm, sum_out, h_v, idx_v, acc_sh):
    cid = lax.axis_index("c")
    sid = lax.axis_index("s")
    wid = cid * NS + sid

    sb = pl.multiple_of(sid * SL, 8)
    pltpu.sync_copy(zsum_hbm.at[pl.ds(sb, SL)], acc_sh.at[pl.ds(sb, SL)])

    @pl.when(sid == NS - 1)
    def _():
        tb = SL * NS
        pltpu.sync_copy(zsum_hbm.at[pl.ds(tb, S_TAIL)], acc_sh.at[pl.ds(tb, S_TAIL)])

    plsc.subcore_barrier()

    def _block(k, _):
        b = wid + NW * k

        @pl.when(b < P1_BLOCKS)
        def _():
            pltpu.sync_copy(ids_hbm.at[b], idx_v)
            rb = pl.multiple_of(b * P1_ROWS, 8)
            pltpu.sync_copy(h_hbm.at[pl.ds(rb, P1_ROWS)], h_v)
            for j in range(P1_SUB):
                pltpu.sync_copy(h_v.at[pl.ds(j * 128, 128)],
                                acc_sh.at[idx_v.at[j]], add=True)
        return 0

    lax.fori_loop(0, P1_ITERS, _block, 0)
    plsc.subcore_barrier()

    pltpu.sync_copy(acc_sh.at[pl.ds(sb, SL)], sum_out.at[cid, pl.ds(sb, SL)])

    @pl.when(sid == NS - 1)
    def _():
        tb = SL * NS
        pltpu.sync_copy(acc_sh.at[pl.ds(tb, S_TAIL)],
                        sum_out.at[cid, pl.ds(tb, S_TAIL)])


_phase1 = functools.partial(
    pl.kernel,
    out_type=jax.ShapeDtypeStruct((NC, S, D), jnp.float32),
    mesh=plsc.VectorSubcoreMesh(core_axis_name="c", subcore_axis_name="s"),
    scratch_types=[
        pltpu.VMEM((P1_ROWS, D), jnp.float32),
        pltpu.VMEM((P1_SUB, 128), jnp.int32),
        pltpu.VMEM_SHARED((S, D), jnp.float32),
    ],
)(_p1_body)


def _mlp_body(sum_ref, cnt_ref, w1t_ref, b1_ref, w2t_ref, b2_ref, g_ref):
    seg = sum_ref[0] + sum_ref[1]
    cnt = cnt_ref[0, :, 0:1] + cnt_ref[1, :, 0:1]
    c = seg / jnp.maximum(cnt, 1.0)
    h = jnp.dot(c, w1t_ref[...], preferred_element_type=jnp.float32)
    h = jnp.maximum(h + b1_ref[...], 0.0)
    h2 = jnp.dot(h, w2t_ref[...], preferred_element_type=jnp.float32)
    h2 = h2 + b2_ref[...]
    g_ref[...] = 1.0 / (1.0 + jnp.exp(-h2))


_phase2 = pl.pallas_call(
    _mlp_body,
    out_shape=jax.ShapeDtypeStruct((S, D), jnp.float32),
)


P3_ROWS = 128
P3_BLOCKS = N // P3_ROWS
P3_ITERS = (P3_BLOCKS + NW - 1) // NW


def _p3_body(h_hbm, ids_hbm, g_hbm, out_hbm, h_v, g_v, idx_v, g_sh):
    cid = lax.axis_index("c")
    sid = lax.axis_index("s")
    wid = cid * NS + sid

    sb = pl.multiple_of(sid * SL, 8)
    pltpu.sync_copy(g_hbm.at[pl.ds(sb, SL)], g_sh.at[pl.ds(sb, SL)])

    @pl.when(sid == NS - 1)
    def _():
        tb = SL * NS
        pltpu.sync_copy(g_hbm.at[pl.ds(tb, S_TAIL)], g_sh.at[pl.ds(tb, S_TAIL)])

    plsc.subcore_barrier()

    def _block(k, _):
        b = wid + NW * k

        @pl.when(b < P3_BLOCKS)
        def _():
            base = pl.multiple_of(b * P3_ROWS, 8)
            pltpu.sync_copy(ids_hbm.at[b], idx_v)
            pltpu.sync_copy(h_hbm.at[pl.ds(base, P3_ROWS)], h_v)
            pltpu.sync_copy(g_sh.at[idx_v], g_v)

            def _row(r, _):
                for j in range(8):
                    sl = pl.ds(j * 16, 16)
                    g_v[r, sl] = h_v[r, sl] * g_v[r, sl]
                return 0

            lax.fori_loop(0, P3_ROWS, _row, 0)
            pltpu.sync_copy(g_v, out_hbm.at[pl.ds(base, P3_ROWS)])
        return 0

    lax.fori_loop(0, P3_ITERS, _block, 0)


_phase3 = functools.partial(
    pl.kernel,
    out_type=jax.ShapeDtypeStruct((N, D), jnp.float32),
    mesh=plsc.VectorSubcoreMesh(core_axis_name="c", subcore_axis_name="s"),
    scratch_types=[
        pltpu.VMEM((P3_ROWS, D), jnp.float32),
        pltpu.VMEM((P3_ROWS, D), jnp.float32),
        pltpu.VMEM((P3_ROWS,), jnp.int32),
        pltpu.VMEM_SHARED((S, D), jnp.float32),
    ],
)(_p3_body)


def kernel(h_V, batch_id, W1, b1, W2, b2):
    ids = batch_id.astype(jnp.int32)
    ids_p0 = ids.reshape(N // 128, 128)
    ids_p1 = ids.reshape(P1_BLOCKS, P1_SUB, 128)
    ids_p3 = ids.reshape(P3_BLOCKS, 128)
    zsum = jnp.zeros((S, D), jnp.float32)
    zcnt = jnp.zeros((S, 16), jnp.float32)
    cnt_p = _phase0(ids_p0, zcnt)
    sum_p = _phase1(h_V, ids_p1, zsum)
    g = _phase2(sum_p, cnt_p, W1.T, b1.reshape(1, D), W2.T, b2.reshape(1, D))
    return _phase3(h_V, ids_p3, g)
